# baseline (speedup 1.0000x reference)
import jax
import jax.numpy as jnp
from jax import lax
from jax.experimental import pallas as pl
from jax.experimental.pallas import tpu as pltpu
from jax.experimental.pallas import tpu_sc as plsc

L = 16
QV = 4
NUM_CORES = 2
NUM_SUBCORES = 16
NUM_TILES = NUM_CORES * NUM_SUBCORES


def _chamfer_body(q_hbm, k_hbm, out_hbm,
                  kpx, kpy, kpz, kn, qsx, qsy, qsz, o_v):
    n_points = kpx.shape[0]
    chunk_len = o_v.shape[0]
    chunks = n_points // chunk_len
    wid = lax.axis_index("s") * NUM_CORES + lax.axis_index("c")
    combo = wid // chunks
    chunk = wid % chunks
    qbase = chunk * chunk_len

    kbase = combo * 3 * n_points
    pltpu.sync_copy(k_hbm.at[pl.ds(kbase, n_points)], kpx)
    pltpu.sync_copy(k_hbm.at[pl.ds(kbase + n_points, n_points)], kpy)
    pltpu.sync_copy(k_hbm.at[pl.ds(kbase + 2 * n_points, n_points)], kpz)
    pltpu.sync_copy(q_hbm.at[pl.ds(kbase + qbase, chunk_len)], qsx)
    pltpu.sync_copy(q_hbm.at[pl.ds(kbase + n_points + qbase, chunk_len)],
                    qsy)
    pltpu.sync_copy(q_hbm.at[pl.ds(kbase + 2 * n_points + qbase, chunk_len)],
                    qsz)

    def pre_body(t, carry):
        sl = pl.ds(t * L, L)
        kx = kpx[sl]
        ky = kpy[sl]
        kz = kpz[sl]
        kn[sl] = kx * kx + ky * ky + kz * kz
        kpx[sl] = kx * (-2.0)
        kpy[sl] = ky * (-2.0)
        kpz[sl] = kz * (-2.0)
        return carry

    lax.fori_loop(0, n_points // L, pre_body, 0)

    def group_body(g, carry):
        base = g * (QV * L)
        qx = [qsx[pl.ds(base + i * L, L)] for i in range(QV)]
        qy = [qsy[pl.ds(base + i * L, L)] for i in range(QV)]
        qz = [qsz[pl.ds(base + i * L, L)] for i in range(QV)]
        qn = [qx[i] * qx[i] + qy[i] * qy[i] + qz[i] * qz[i]
              for i in range(QV)]
        inf = jnp.full((L,), jnp.inf, jnp.float32)

        def key_body(t, accs):
            sl = pl.ds(t * L, L)
            kxv = kpx[sl]
            kyv = kpy[sl]
            kzv = kpz[sl]
            knv = kn[sl]
            accs = list(accs)
            for u in range(L):
                idx = jnp.full((L,), u, jnp.int32)
                bkx = kxv.at[idx].get(mode="promise_in_bounds")
                bky = kyv.at[idx].get(mode="promise_in_bounds")
                bkz = kzv.at[idx].get(mode="promise_in_bounds")
                bkn = knv.at[idx].get(mode="promise_in_bounds")
                for i in range(QV):
                    d = qx[i] * bkx + qy[i] * bky + qz[i] * bkz + bkn
                    accs[i] = jnp.minimum(accs[i], d)
            return tuple(accs)

        accs = lax.fori_loop(0, n_points // L, key_body, (inf,) * QV)
        for i in range(QV):
            o_v[pl.ds(base + i * L, L)] = accs[i] + qn[i]
        return carry

    lax.fori_loop(0, chunk_len // (QV * L), group_body, 0)

    pltpu.sync_copy(o_v, out_hbm.at[pl.ds(wid * chunk_len, chunk_len)])


@jax.jit
def _chamfer_sc(q, k):
    n_items, _, n_points = q.shape
    mesh = plsc.VectorSubcoreMesh(core_axis_name="c", subcore_axis_name="s",
                                  num_cores=NUM_CORES,
                                  num_subcores=NUM_SUBCORES)
    chunk_len = n_points * n_items // NUM_TILES
    return pl.kernel(
        _chamfer_body,
        out_type=jax.ShapeDtypeStruct((n_items * n_points,), jnp.float32),
        mesh=mesh,
        scratch_types=[
            pltpu.VMEM((n_points,), jnp.float32),
            pltpu.VMEM((n_points,), jnp.float32),
            pltpu.VMEM((n_points,), jnp.float32),
            pltpu.VMEM((n_points,), jnp.float32),
            pltpu.VMEM((chunk_len,), jnp.float32),
            pltpu.VMEM((chunk_len,), jnp.float32),
            pltpu.VMEM((chunk_len,), jnp.float32),
            pltpu.VMEM((chunk_len,), jnp.float32),
        ],
    )(q.reshape(-1), k.reshape(-1))


def kernel(input1, input2):
    x1t = jnp.transpose(input1, (0, 2, 1))
    x2t = jnp.transpose(input2, (0, 2, 1))
    q = jnp.concatenate([x1t, x2t], axis=0)
    k = jnp.concatenate([x2t, x1t], axis=0)
    out = _chamfer_sc(q, k).reshape(8, -1)
    return out[:4], out[4:]
